# Initial kernel scaffold; baseline (speedup 1.0000x reference)
import jax
import jax.numpy as jnp
from jax.experimental import pallas as pl


def kernel(output, mask, ind, target):
    raise NotImplementedError("write your pallas kernel here")


# speedup vs baseline: 1.3212x; 1.3212x over previous
import functools

import jax
import jax.numpy as jnp
from jax import lax
from jax.experimental import pallas as pl
from jax.experimental.pallas import tpu as pltpu
from jax.experimental.pallas import tpu_sc as plsc

_B, _C, _H, _W, _K = 16, 2, 128, 128, 128
_HW = _H * _W
_L = 16
_PAD = 128


def _loss_body(outflat, ind, maskf, targf, out,
               ind_v, idx0_v, idx1_v, pred0_v, pred1_v,
               mask_v, targ_v, partl_v, partm_v, gath_v, out_v, shared, sem):
    cid = lax.axis_index("c")
    sid = lax.axis_index("s")
    b = sid

    pltpu.sync_copy(ind.at[b], ind_v)
    pltpu.sync_copy(maskf.at[b], mask_v)
    pltpu.sync_copy(targf.at[b], targ_v)

    base0 = (2 * b) * _HW
    for i in range(_K // _L):
        chunk = ind_v[pl.ds(i * _L, _L)]
        idx0_v[pl.ds(i * _L, _L)] = chunk + base0
        idx1_v[pl.ds(i * _L, _L)] = chunk + (base0 + _HW)

    d0 = pltpu.async_copy(outflat.at[idx0_v], pred0_v, sem)
    d1 = pltpu.async_copy(outflat.at[idx1_v], pred1_v, sem)
    d0.wait()
    d1.wait()

    accl = jnp.zeros((_L,), jnp.float32)
    accm = jnp.zeros((_L,), jnp.float32)
    for i in range(_K * _C // _L):
        p = (pred0_v if i < 8 else pred1_v)[pl.ds((i % 8) * _L, _L)]
        m = mask_v[pl.ds(i * _L, _L)]
        t = targ_v[pl.ds(i * _L, _L)]
        accl = accl + jnp.abs(p * m - t * m)
        accm = accm + m

    partl_v[...] = accl
    partm_v[...] = accm
    pltpu.sync_copy(partl_v, shared.at[_PAD + b])
    pltpu.sync_copy(partm_v, shared.at[_PAD + _B + b])
    plsc.subcore_barrier()

    @pl.when((cid == 0) & (sid == 0))
    def _finalize():
        pltpu.sync_copy(shared.at[pl.ds(_PAD, 2 * _B)], gath_v)
        suml = jnp.zeros((_L,), jnp.float32)
        summ = jnp.zeros((_L,), jnp.float32)
        for i in range(_B):
            suml = suml + gath_v[i, :]
            summ = summ + gath_v[_B + i, :]
        sl = jnp.sum(suml)
        sm = jnp.sum(summ)
        num = jnp.full((_L,), sl, jnp.float32)
        den = jnp.full((_L,), sm, jnp.float32) + jnp.float32(1e-4)
        out_v[...] = num / den
        pltpu.sync_copy(out_v, out)


_sc_loss = functools.partial(
    pl.kernel,
    mesh=plsc.VectorSubcoreMesh(core_axis_name="c", subcore_axis_name="s"),
    compiler_params=pltpu.CompilerParams(needs_layout_passes=False),
    out_type=jax.ShapeDtypeStruct((_L,), jnp.float32),
    scratch_types=[
        pltpu.VMEM((_K,), jnp.int32),
        pltpu.VMEM((2 * _L * 4,), jnp.int32),
        pltpu.VMEM((2 * _L * 4,), jnp.int32),
        pltpu.VMEM((2 * _L * 4,), jnp.float32),
        pltpu.VMEM((2 * _L * 4,), jnp.float32),
        pltpu.VMEM((_K * _C,), jnp.float32),
        pltpu.VMEM((_K * _C,), jnp.float32),
        pltpu.VMEM((_L,), jnp.float32),
        pltpu.VMEM((_L,), jnp.float32),
        pltpu.VMEM((2 * _B, _L), jnp.float32),
        pltpu.VMEM((_L,), jnp.float32),
        pltpu.VMEM_SHARED((_PAD + 2 * _B, _L), jnp.float32),
        pltpu.SemaphoreType.DMA,
    ],
)(_loss_body)


def kernel(output, mask, ind, target):
    B, C, H, W = output.shape
    K = ind.shape[1]
    assert (B, C, H, W, K) == (_B, _C, _H, _W, _K)
    outflat = output.reshape(B * C * H * W)
    maskf = mask.transpose(0, 2, 1).reshape(B, C * K)
    targf = target.transpose(0, 2, 1).reshape(B, C * K)
    res = _sc_loss(outflat, ind, maskf, targf)
    return res[0]


# speedup vs baseline: 1.3549x; 1.0255x over previous
import functools

import jax
import jax.numpy as jnp
from jax import lax
from jax.experimental import pallas as pl
from jax.experimental.pallas import tpu as pltpu
from jax.experimental.pallas import tpu_sc as plsc

_B, _C, _H, _W, _K = 16, 2, 128, 128, 128
_HW = _H * _W
_L = 16
_PAD = 128


def _loss_body(outflat, ind, maskf, targf, out,
               ind_v, idx0_v, idx1_v, pred0_v, pred1_v,
               mask_v, targ_v, partl_v, partm_v, gath_v, out_v, shared,
               sem_i, sem_m, sem_t, sem_g):
    cid = lax.axis_index("c")
    sid = lax.axis_index("s")
    b = sid

    di = pltpu.async_copy(ind.at[b], ind_v, sem_i)
    dm = pltpu.async_copy(maskf.at[b], mask_v, sem_m)
    dt = pltpu.async_copy(targf.at[b], targ_v, sem_t)
    di.wait()

    base0 = (2 * b) * _HW
    iota = lax.broadcasted_iota(jnp.int32, (_L,), 0)
    kidx0 = iota // 2
    choff = (iota % 2) * _HW
    for j in range(_K * _C // _L):
        vals = plsc.load_gather(ind_v, [8 * j + kidx0])
        chunk = vals + (choff + base0)
        if j < 8:
            idx0_v[pl.ds(j * _L, _L)] = chunk
        else:
            idx1_v[pl.ds((j - 8) * _L, _L)] = chunk

    d0 = pltpu.async_copy(outflat.at[idx0_v], pred0_v, sem_g)
    d1 = pltpu.async_copy(outflat.at[idx1_v], pred1_v, sem_g)
    dm.wait()
    dt.wait()
    d0.wait()
    d1.wait()

    accl = jnp.zeros((_L,), jnp.float32)
    accm = jnp.zeros((_L,), jnp.float32)
    for i in range(_K * _C // _L):
        p = (pred0_v if i < 8 else pred1_v)[pl.ds((i % 8) * _L, _L)]
        m = mask_v[pl.ds(i * _L, _L)]
        t = targ_v[pl.ds(i * _L, _L)]
        accl = accl + jnp.abs(p * m - t * m)
        accm = accm + m

    partl_v[...] = accl
    partm_v[...] = accm
    pltpu.sync_copy(partl_v, shared.at[_PAD + b])
    pltpu.sync_copy(partm_v, shared.at[_PAD + _B + b])
    plsc.subcore_barrier()

    @pl.when((cid == 0) & (sid == 0))
    def _finalize():
        pltpu.sync_copy(shared.at[pl.ds(_PAD, 2 * _B)], gath_v)
        suml = jnp.zeros((_L,), jnp.float32)
        summ = jnp.zeros((_L,), jnp.float32)
        for i in range(_B):
            suml = suml + gath_v[i, :]
            summ = summ + gath_v[_B + i, :]
        sl = jnp.sum(suml)
        sm = jnp.sum(summ)
        num = jnp.full((_L,), sl, jnp.float32)
        den = jnp.full((_L,), sm, jnp.float32) + jnp.float32(1e-4)
        out_v[...] = num / den
        pltpu.sync_copy(out_v, out)


_sc_loss = functools.partial(
    pl.kernel,
    mesh=plsc.VectorSubcoreMesh(core_axis_name="c", subcore_axis_name="s"),
    compiler_params=pltpu.CompilerParams(needs_layout_passes=False),
    out_type=jax.ShapeDtypeStruct((_L,), jnp.float32),
    scratch_types=[
        pltpu.VMEM((_K,), jnp.int32),
        pltpu.VMEM((_K * _C // 2,), jnp.int32),
        pltpu.VMEM((_K * _C // 2,), jnp.int32),
        pltpu.VMEM((_K * _C // 2,), jnp.float32),
        pltpu.VMEM((_K * _C // 2,), jnp.float32),
        pltpu.VMEM((_K * _C,), jnp.float32),
        pltpu.VMEM((_K * _C,), jnp.float32),
        pltpu.VMEM((_L,), jnp.float32),
        pltpu.VMEM((_L,), jnp.float32),
        pltpu.VMEM((2 * _B, _L), jnp.float32),
        pltpu.VMEM((_L,), jnp.float32),
        pltpu.VMEM_SHARED((_PAD + 2 * _B, _L), jnp.float32),
        pltpu.SemaphoreType.DMA,
        pltpu.SemaphoreType.DMA,
        pltpu.SemaphoreType.DMA,
        pltpu.SemaphoreType.DMA,
    ],
)(_loss_body)


def kernel(output, mask, ind, target):
    B, C, H, W = output.shape
    K = ind.shape[1]
    assert (B, C, H, W, K) == (_B, _C, _H, _W, _K)
    outflat = output.reshape(B * C * H * W)
    maskf = mask.reshape(B, K * C)
    targf = target.reshape(B, K * C)
    res = _sc_loss(outflat, ind, maskf, targf)
    return res[0]


# speedup vs baseline: 1.3564x; 1.0012x over previous
import functools

import jax
import jax.numpy as jnp
from jax import lax
from jax.experimental import pallas as pl
from jax.experimental.pallas import tpu as pltpu
from jax.experimental.pallas import tpu_sc as plsc

_B, _C, _H, _W, _K = 16, 2, 128, 128, 128
_HW = _H * _W
_L = 16
_PAD = 128


def _loss_body(outflat, ind, maskf, targf, out,
               ind_v, idx0_v, idx1_v, pred0_v, pred1_v,
               mask_v, targ_v, partl_v, partm_v, gath_v, out_v, shared,
               sem_i, sem_m, sem_t, sem_g):
    cid = lax.axis_index("c")
    sid = lax.axis_index("s")
    b = sid

    @pl.when(cid == 0)
    def _core0():
        _tile_work(outflat, ind, maskf, targf, out, b, sid,
                   ind_v, idx0_v, idx1_v, pred0_v, pred1_v,
                   mask_v, targ_v, partl_v, partm_v, gath_v, out_v, shared,
                   sem_i, sem_m, sem_t, sem_g)


def _tile_work(outflat, ind, maskf, targf, out, b, sid,
               ind_v, idx0_v, idx1_v, pred0_v, pred1_v,
               mask_v, targ_v, partl_v, partm_v, gath_v, out_v, shared,
               sem_i, sem_m, sem_t, sem_g):
    di = pltpu.async_copy(ind.at[b], ind_v, sem_i)
    dm = pltpu.async_copy(maskf.at[b], mask_v, sem_m)
    dt = pltpu.async_copy(targf.at[b], targ_v, sem_t)
    di.wait()

    base0 = (2 * b) * _HW
    iota = lax.broadcasted_iota(jnp.int32, (_L,), 0)
    kidx0 = iota // 2
    choff = (iota % 2) * _HW
    for j in range(_K * _C // _L):
        vals = plsc.load_gather(ind_v, [8 * j + kidx0])
        chunk = vals + (choff + base0)
        if j < 8:
            idx0_v[pl.ds(j * _L, _L)] = chunk
        else:
            idx1_v[pl.ds((j - 8) * _L, _L)] = chunk

    d0 = pltpu.async_copy(outflat.at[idx0_v], pred0_v, sem_g)
    d1 = pltpu.async_copy(outflat.at[idx1_v], pred1_v, sem_g)
    dm.wait()
    dt.wait()
    d0.wait()
    d1.wait()

    accl = jnp.zeros((_L,), jnp.float32)
    accm = jnp.zeros((_L,), jnp.float32)
    for i in range(_K * _C // _L):
        p = (pred0_v if i < 8 else pred1_v)[pl.ds((i % 8) * _L, _L)]
        m = mask_v[pl.ds(i * _L, _L)]
        t = targ_v[pl.ds(i * _L, _L)]
        accl = accl + jnp.abs(p * m - t * m)
        accm = accm + m

    partl_v[...] = accl
    partm_v[...] = accm
    pltpu.sync_copy(partl_v, shared.at[_PAD + b])
    pltpu.sync_copy(partm_v, shared.at[_PAD + _B + b])
    plsc.subcore_barrier()

    @pl.when(sid == 0)
    def _finalize():
        pltpu.sync_copy(shared.at[pl.ds(_PAD, 2 * _B)], gath_v)
        suml = jnp.zeros((_L,), jnp.float32)
        summ = jnp.zeros((_L,), jnp.float32)
        for i in range(_B):
            suml = suml + gath_v[i, :]
            summ = summ + gath_v[_B + i, :]
        sl = jnp.sum(suml)
        sm = jnp.sum(summ)
        num = jnp.full((_L,), sl, jnp.float32)
        den = jnp.full((_L,), sm, jnp.float32) + jnp.float32(1e-4)
        out_v[...] = num / den
        pltpu.sync_copy(out_v, out)


_sc_loss = functools.partial(
    pl.kernel,
    mesh=plsc.VectorSubcoreMesh(core_axis_name="c", subcore_axis_name="s"),
    compiler_params=pltpu.CompilerParams(needs_layout_passes=False),
    out_type=jax.ShapeDtypeStruct((_L,), jnp.float32),
    scratch_types=[
        pltpu.VMEM((_K,), jnp.int32),
        pltpu.VMEM((_K * _C // 2,), jnp.int32),
        pltpu.VMEM((_K * _C // 2,), jnp.int32),
        pltpu.VMEM((_K * _C // 2,), jnp.float32),
        pltpu.VMEM((_K * _C // 2,), jnp.float32),
        pltpu.VMEM((_K * _C,), jnp.float32),
        pltpu.VMEM((_K * _C,), jnp.float32),
        pltpu.VMEM((_L,), jnp.float32),
        pltpu.VMEM((_L,), jnp.float32),
        pltpu.VMEM((2 * _B, _L), jnp.float32),
        pltpu.VMEM((_L,), jnp.float32),
        pltpu.VMEM_SHARED((_PAD + 2 * _B, _L), jnp.float32),
        pltpu.SemaphoreType.DMA,
        pltpu.SemaphoreType.DMA,
        pltpu.SemaphoreType.DMA,
        pltpu.SemaphoreType.DMA,
    ],
)(_loss_body)


def kernel(output, mask, ind, target):
    B, C, H, W = output.shape
    K = ind.shape[1]
    assert (B, C, H, W, K) == (_B, _C, _H, _W, _K)
    outflat = output.reshape(B * C * H * W)
    maskf = mask.reshape(B, K * C)
    targf = target.reshape(B, K * C)
    res = _sc_loss(outflat, ind, maskf, targf)
    return res[0]


# speedup vs baseline: 1.3673x; 1.0080x over previous
import functools

import jax
import jax.numpy as jnp
from jax import lax
from jax.experimental import pallas as pl
from jax.experimental.pallas import tpu as pltpu
from jax.experimental.pallas import tpu_sc as plsc

_B, _C, _H, _W, _K = 16, 2, 128, 128, 128
_HW = _H * _W
_L = 16
_PAD = 128


def _loss_body(outflat, ind, maskf, targf, out,
               ind_v, idx0_v, idx1_v, pred0_v, pred1_v,
               mask_v, targ_v, part2_v, gath_v, out_v, shared,
               sem_i, sem_m, sem_t, sem_g):
    cid = lax.axis_index("c")
    sid = lax.axis_index("s")
    b = sid

    @pl.when(cid == 0)
    def _core0():
        _tile_work(outflat, ind, maskf, targf, out, b, sid,
                   ind_v, idx0_v, idx1_v, pred0_v, pred1_v,
                   mask_v, targ_v, part2_v, gath_v, out_v, shared,
                   sem_i, sem_m, sem_t, sem_g)


def _tile_work(outflat, ind, maskf, targf, out, b, sid,
               ind_v, idx0_v, idx1_v, pred0_v, pred1_v,
               mask_v, targ_v, part2_v, gath_v, out_v, shared,
               sem_i, sem_m, sem_t, sem_g):
    di = pltpu.async_copy(ind.at[b], ind_v, sem_i)
    dm = pltpu.async_copy(maskf.at[b], mask_v, sem_m)
    dt = pltpu.async_copy(targf.at[b], targ_v, sem_t)
    di.wait()

    base0 = (2 * b) * _HW
    iota = lax.broadcasted_iota(jnp.int32, (_L,), 0)
    kidx0 = iota // 2
    choff = (iota % 2) * _HW
    for j in range(_K * _C // _L):
        vals = plsc.load_gather(ind_v, [8 * j + kidx0])
        chunk = vals + (choff + base0)
        if j < 8:
            idx0_v[pl.ds(j * _L, _L)] = chunk
        else:
            idx1_v[pl.ds((j - 8) * _L, _L)] = chunk

    d0 = pltpu.async_copy(outflat.at[idx0_v], pred0_v, sem_g)
    d1 = pltpu.async_copy(outflat.at[idx1_v], pred1_v, sem_g)
    dm.wait()
    dt.wait()
    d0.wait()
    d1.wait()

    accl = jnp.zeros((_L,), jnp.float32)
    accm = jnp.zeros((_L,), jnp.float32)
    for i in range(_K * _C // _L):
        p = (pred0_v if i < 8 else pred1_v)[pl.ds((i % 8) * _L, _L)]
        m = mask_v[pl.ds(i * _L, _L)]
        t = targ_v[pl.ds(i * _L, _L)]
        accl = accl + jnp.abs(p * m - t * m)
        accm = accm + m

    part2_v[0, :] = accl
    part2_v[1, :] = accm
    pltpu.sync_copy(part2_v, shared.at[pl.ds(_PAD + 2 * b, 2)])
    plsc.subcore_barrier()

    @pl.when(sid == 0)
    def _finalize():
        pltpu.sync_copy(shared.at[pl.ds(_PAD, 2 * _B)], gath_v)
        suml = jnp.zeros((_L,), jnp.float32)
        summ = jnp.zeros((_L,), jnp.float32)
        for i in range(_B):
            suml = suml + gath_v[2 * i, :]
            summ = summ + gath_v[2 * i + 1, :]
        sl = jnp.sum(suml)
        sm = jnp.sum(summ)
        num = jnp.full((_L,), sl, jnp.float32)
        den = jnp.full((_L,), sm, jnp.float32) + jnp.float32(1e-4)
        out_v[...] = num / den
        pltpu.sync_copy(out_v, out)


_sc_loss = functools.partial(
    pl.kernel,
    mesh=plsc.VectorSubcoreMesh(core_axis_name="c", subcore_axis_name="s"),
    compiler_params=pltpu.CompilerParams(needs_layout_passes=False),
    out_type=jax.ShapeDtypeStruct((_L,), jnp.float32),
    scratch_types=[
        pltpu.VMEM((_K,), jnp.int32),
        pltpu.VMEM((_K * _C // 2,), jnp.int32),
        pltpu.VMEM((_K * _C // 2,), jnp.int32),
        pltpu.VMEM((_K * _C // 2,), jnp.float32),
        pltpu.VMEM((_K * _C // 2,), jnp.float32),
        pltpu.VMEM((_K * _C,), jnp.float32),
        pltpu.VMEM((_K * _C,), jnp.float32),
        pltpu.VMEM((2, _L), jnp.float32),
        pltpu.VMEM((2 * _B, _L), jnp.float32),
        pltpu.VMEM((_L,), jnp.float32),
        pltpu.VMEM_SHARED((_PAD + 2 * _B, _L), jnp.float32),
        pltpu.SemaphoreType.DMA,
        pltpu.SemaphoreType.DMA,
        pltpu.SemaphoreType.DMA,
        pltpu.SemaphoreType.DMA,
    ],
)(_loss_body)


def kernel(output, mask, ind, target):
    B, C, H, W = output.shape
    K = ind.shape[1]
    assert (B, C, H, W, K) == (_B, _C, _H, _W, _K)
    outflat = output.reshape(B * C * H * W)
    maskf = mask.reshape(B, K * C)
    targf = target.reshape(B, K * C)
    res = _sc_loss(outflat, ind, maskf, targf)
    return res[0]
